# baseline (speedup 1.0000x reference)
import functools

import jax
import jax.numpy as jnp
from jax import lax
from jax.experimental import pallas as pl
from jax.experimental.pallas import tpu as pltpu
from jax.experimental.pallas import tpu_sc as plsc

N = 10000
E = 320000
H = 128
NC = 2
NS = 16
NW = NC * NS
EPW = E // NW
KC = 40
G = 10
NG = EPW // (KC * G)
RPT = 624
RTAIL = N - NS * RPT
LANES = H // 16

BN = 1000


def _enc_body(x_ref, we_ref, be_ref, ws_ref, wd_ref, bedge_ref,
              h_ref, ab_ref):
    h = jnp.dot(x_ref[...], we_ref[...],
                preferred_element_type=jnp.float32) + be_ref[...]
    h_ref[...] = h
    ab_ref[0] = jnp.dot(h, ws_ref[...], preferred_element_type=jnp.float32)
    ab_ref[1] = jnp.dot(h, wd_ref[...],
                        preferred_element_type=jnp.float32) + bedge_ref[...]


def _encode(x, W_enc, b_enc, Ws, Wd, b_edge):
    nb = N // BN
    full = lambda shape: pl.BlockSpec(shape, lambda i: (0, 0))
    return pl.pallas_call(
        _enc_body,
        grid=(nb,),
        in_specs=[
            pl.BlockSpec((BN, 128), lambda i: (i, 0)),
            full((128, H)), full((1, H)), full((H, H)), full((H, H)),
            full((1, H)),
        ],
        out_specs=[pl.BlockSpec((BN, H), lambda i: (i, 0)),
                   pl.BlockSpec((2, BN, H), lambda i: (0, i, 0))],
        out_shape=[jax.ShapeDtypeStruct((N, H), jnp.float32),
                   jax.ShapeDtypeStruct((2, N, H), jnp.float32)],
    )(x, W_enc, b_enc, Ws, Wd, b_edge)


def _edge_c_body(ea_ref, we_ref, c_ref):
    c_ref[...] = jnp.dot(ea_ref[...], we_ref[...],
                         preferred_element_type=jnp.float32)


def _edge_c(edge_attr, We):
    BE = 4000
    return pl.pallas_call(
        _edge_c_body,
        grid=(E // BE,),
        in_specs=[pl.BlockSpec((BE, 16), lambda i: (i, 0)),
                  pl.BlockSpec((16, H), lambda i: (0, 0))],
        out_specs=pl.BlockSpec((BE, H), lambda i: (i, 0)),
        out_shape=jax.ShapeDtypeStruct((E, H), jnp.float32),
    )(edge_attr, We)


def _node_body(h_ref, agg_ref, wn1_ref, wn2_ref, bn_ref, ws_ref, wd_ref,
               bedge_ref, h2_ref, ab_ref):
    ag = agg_ref[0] + agg_ref[1]
    hn = (jnp.dot(h_ref[...], wn1_ref[...], preferred_element_type=jnp.float32)
          + jnp.dot(ag, wn2_ref[...], preferred_element_type=jnp.float32)
          + bn_ref[...])
    hn = jnp.maximum(hn, 0.0)
    h2_ref[...] = hn
    ab_ref[0] = jnp.dot(hn, ws_ref[...], preferred_element_type=jnp.float32)
    ab_ref[1] = jnp.dot(hn, wd_ref[...],
                        preferred_element_type=jnp.float32) + bedge_ref[...]


def _node_update(h, agg, Wn1, Wn2, b_node, Ws, Wd, b_edge):
    nb = N // BN
    full = lambda shape: pl.BlockSpec(shape, lambda i: tuple(0 for _ in shape))
    return pl.pallas_call(
        _node_body,
        grid=(nb,),
        in_specs=[
            pl.BlockSpec((BN, H), lambda i: (i, 0)),
            pl.BlockSpec((NC, BN, H), lambda i: (0, i, 0)),
            full((H, H)), full((H, H)), full((1, H)),
            full((H, H)), full((H, H)), full((1, H)),
        ],
        out_specs=[pl.BlockSpec((BN, H), lambda i: (i, 0)),
                   pl.BlockSpec((2, BN, H), lambda i: (0, i, 0))],
        out_shape=[jax.ShapeDtypeStruct((N, H), jnp.float32),
                   jax.ShapeDtypeStruct((2, N, H), jnp.float32)],
    )(h, agg, Wn1, Wn2, b_node, Ws, Wd, b_edge)


def _dec_body(h_ref, w1_ref, b1_ref, w2_ref, b2_ref, o_ref):
    t = jnp.dot(h_ref[...], w1_ref[...],
                preferred_element_type=jnp.float32) + b1_ref[...]
    t = jnp.maximum(t, 0.0)
    o_ref[...] = jnp.dot(t, w2_ref[...],
                         preferred_element_type=jnp.float32) + b2_ref[...]


def _decode(h, W1c, b1c, W2blk, b2c, d_out):
    nb = N // BN
    full = lambda shape: pl.BlockSpec(shape, lambda i: (0, 0))
    return pl.pallas_call(
        _dec_body,
        grid=(nb,),
        in_specs=[
            pl.BlockSpec((BN, H), lambda i: (i, 0)),
            full((H, W1c.shape[1])), full((1, W1c.shape[1])),
            full((W2blk.shape[0], d_out)), full((1, d_out)),
        ],
        out_specs=pl.BlockSpec((BN, d_out), lambda i: (i, 0)),
        out_shape=jax.ShapeDtypeStruct((N, d_out), jnp.float32),
    )(h, W1c, b1c, W2blk, b2c)


def _sc_agg_body(t_hbm, c_hbm, cat_hbm, dst_hbm, out_hbm,
                 cat_g, dst_g, ab0, cv0, ab1, cv1, agg_sp,
                 sem, isem, ssem):
    c = lax.axis_index("c")
    s = lax.axis_index("s")
    w = c * NS + s
    ebase = w * EPW
    bufs = ((ab0, cv0), (ab1, cv1))

    def _zrow(r, carry):
        for j in range(LANES):
            ab0[r, pl.ds(j * 16, 16)] = jnp.zeros((16,), jnp.float32)
        return carry
    lax.fori_loop(0, KC, _zrow, 0)
    rbase = pl.multiple_of(s * RPT, 8)
    off = 0
    while off < RPT:
        n = min(KC, RPT - off)
        pltpu.sync_copy(ab0.at[pl.ds(0, n)], agg_sp.at[pl.ds(rbase + off, n)])
        off += n

    @pl.when(s == NS - 1)
    def _zero_tail():
        pltpu.sync_copy(ab0.at[pl.ds(0, RTAIL)],
                        agg_sp.at[pl.ds(NS * RPT, RTAIL)])
    plsc.subcore_barrier()

    def _group(g, carry):
        gs = pltpu.make_async_copy(cat_hbm.at[w, g], cat_g, isem)
        gd = pltpu.make_async_copy(dst_hbm.at[w, g], dst_g, isem)
        gs.start()
        gd.start()
        gs.wait()
        gd.wait()

        def _mk(i):
            ab_, cv_ = bufs[i % 2]
            cb = pl.multiple_of(ebase + (g * G + i) * KC, 8)
            return (pltpu.make_async_copy(t_hbm.at[cat_g.at[i]], ab_, sem),
                    pltpu.make_async_copy(c_hbm.at[pl.ds(cb, KC)], cv_, sem))

        def _sc_start(i):
            cv_ = bufs[i % 2][1]
            pltpu.async_copy(cv_, agg_sp.at[dst_g.at[i]], ssem, add=True)

        def _sc_wait(i):
            cv_ = bufs[i % 2][1]
            pltpu.make_async_copy(cv_, agg_sp.at[dst_g.at[i]], ssem).wait()

        for d in _mk(0):
            d.start()
        for i in range(G):
            for d in _mk(i):
                d.wait()
            if i >= 1:
                _sc_wait(i - 1)
            if i + 1 < G:
                for d in _mk(i + 1):
                    d.start()
            ab_, cv_ = bufs[i % 2]

            @plsc.parallel_loop(0, KC, unroll=4)
            def _row(r, ab_=ab_, cv_=cv_):
                for j in range(LANES):
                    sl = pl.ds(j * 16, 16)
                    cv_[r, sl] = jnp.maximum(
                        ab_[r, sl] + ab_[KC + r, sl] + cv_[r, sl], 0.0)
            _sc_start(i)
        _sc_wait(G - 1)
        return carry
    lax.fori_loop(0, NG, _group, 0)

    plsc.subcore_barrier()
    pltpu.sync_copy(agg_sp.at[pl.ds(rbase, RPT)],
                    out_hbm.at[c, pl.ds(rbase, RPT)])

    @pl.when(s == NS - 1)
    def _write_tail():
        pltpu.sync_copy(agg_sp.at[pl.ds(NS * RPT, RTAIL)],
                        out_hbm.at[c, pl.ds(NS * RPT, RTAIL)])


@functools.lru_cache(maxsize=None)
def _make_sc_agg():
    mesh = plsc.VectorSubcoreMesh(core_axis_name="c", subcore_axis_name="s",
                                  num_cores=NC, num_subcores=NS)
    return pl.kernel(
        _sc_agg_body,
        out_type=jax.ShapeDtypeStruct((NC, N, H), jnp.float32),
        mesh=mesh,
        scratch_types=[
            pltpu.VMEM((G, 2 * KC), jnp.int32),
            pltpu.VMEM((G, KC), jnp.int32),
            pltpu.VMEM((2 * KC, H), jnp.float32),
            pltpu.VMEM((KC, H), jnp.float32),
            pltpu.VMEM((2 * KC, H), jnp.float32),
            pltpu.VMEM((KC, H), jnp.float32),
            pltpu.VMEM_SHARED((N, H), jnp.float32),
            pltpu.SemaphoreType.DMA,
            pltpu.SemaphoreType.DMA,
            pltpu.SemaphoreType.DMA,
        ],
    )


def _sc_agg(T, C, cat4, dst4):
    return _make_sc_agg()(T, C, cat4, dst4)


def kernel(x, edge_index, edge_attr, layer_num, W_enc, b_enc, W_edge, b_edge,
           W_node, b_node, dec):
    Ws = W_edge[:H]
    Wd = W_edge[H:2 * H]
    We = W_edge[2 * H:]
    Wn1 = W_node[:H]
    Wn2 = W_node[H:]
    b_enc2 = b_enc.reshape(1, H)
    b_edge2 = b_edge.reshape(1, H)
    b_node2 = b_node.reshape(1, H)

    src4 = edge_index[0].astype(jnp.int32).reshape(NW, NG, G, KC)
    dst4 = edge_index[1].astype(jnp.int32).reshape(NW, NG, G, KC)
    cat4 = jnp.concatenate([src4, dst4 + N], axis=3)

    C = _edge_c(edge_attr, We)
    h, ab = _encode(x, W_enc, b_enc2, Ws, Wd, b_edge2)

    def _layer(_, carry):
        h, ab = carry
        agg = _sc_agg(ab.reshape(2 * N, H), C, cat4, dst4)
        return _node_update(h, agg, Wn1, Wn2, b_node2, Ws, Wd, b_edge2)

    h, _ = lax.fori_loop(0, layer_num, _layer, (h, ab))

    names = ["dispX", "dispZ", "momentY", "momentZ", "shearY", "shearZ"]
    W1c = jnp.concatenate([dec[k][0] for k in names], axis=1)
    b1c = jnp.concatenate([dec[k][1] for k in names]).reshape(1, -1)
    W2blk = jax.scipy.linalg.block_diag(*[dec[k][2] for k in names])
    b2c = jnp.concatenate([dec[k][3] for k in names]).reshape(1, -1)
    return _decode(h, W1c, b1c, W2blk, b2c, W2blk.shape[1])


# speedup vs baseline: 1.0450x; 1.0450x over previous
import functools

import jax
import jax.numpy as jnp
from jax import lax
from jax.experimental import pallas as pl
from jax.experimental.pallas import tpu as pltpu
from jax.experimental.pallas import tpu_sc as plsc

N = 10000
E = 320000
H = 128
NC = 2
NS = 16
NW = NC * NS
EPW = E // NW
KC = 40
G = 25
NG = EPW // (KC * G)
RPT = 624
RTAIL = N - NS * RPT
LANES = H // 16

BN = 2000


def _enc_body(x_ref, we_ref, be_ref, ws_ref, wd_ref, bedge_ref,
              h_ref, a_ref, b_ref):
    h = jnp.dot(x_ref[...], we_ref[...],
                preferred_element_type=jnp.float32) + be_ref[...]
    h_ref[...] = h
    a_ref[...] = jnp.dot(h, ws_ref[...], preferred_element_type=jnp.float32)
    b_ref[...] = jnp.dot(h, wd_ref[...],
                         preferred_element_type=jnp.float32) + bedge_ref[...]


def _encode(x, W_enc, b_enc, Ws, Wd, b_edge):
    nb = N // BN
    full = lambda shape: pl.BlockSpec(shape, lambda i: (0, 0))
    return pl.pallas_call(
        _enc_body,
        grid=(nb,),
        in_specs=[
            pl.BlockSpec((BN, 128), lambda i: (i, 0)),
            full((128, H)), full((1, H)), full((H, H)), full((H, H)),
            full((1, H)),
        ],
        out_specs=[pl.BlockSpec((BN, H), lambda i: (i, 0))] * 3,
        out_shape=[jax.ShapeDtypeStruct((N, H), jnp.float32)] * 3,
    )(x, W_enc, b_enc, Ws, Wd, b_edge)


def _edge_c_body(ea_ref, we_ref, c_ref):
    c_ref[...] = jnp.dot(ea_ref[...], we_ref[...],
                         preferred_element_type=jnp.float32)


def _edge_c(edge_attr, We):
    BE = 4000
    return pl.pallas_call(
        _edge_c_body,
        grid=(E // BE,),
        in_specs=[pl.BlockSpec((BE, 16), lambda i: (i, 0)),
                  pl.BlockSpec((16, H), lambda i: (0, 0))],
        out_specs=pl.BlockSpec((BE, H), lambda i: (i, 0)),
        out_shape=jax.ShapeDtypeStruct((E, H), jnp.float32),
    )(edge_attr, We)


def _node_body(h_ref, agg_ref, wn1_ref, wn2_ref, bn_ref, ws_ref, wd_ref,
               bedge_ref, h2_ref, a_ref, b_ref):
    ag = agg_ref[0] + agg_ref[1]
    hn = (jnp.dot(h_ref[...], wn1_ref[...], preferred_element_type=jnp.float32)
          + jnp.dot(ag, wn2_ref[...], preferred_element_type=jnp.float32)
          + bn_ref[...])
    hn = jnp.maximum(hn, 0.0)
    h2_ref[...] = hn
    a_ref[...] = jnp.dot(hn, ws_ref[...], preferred_element_type=jnp.float32)
    b_ref[...] = jnp.dot(hn, wd_ref[...],
                         preferred_element_type=jnp.float32) + bedge_ref[...]


def _node_update(h, agg, Wn1, Wn2, b_node, Ws, Wd, b_edge):
    nb = N // BN
    full = lambda shape: pl.BlockSpec(shape, lambda i: tuple(0 for _ in shape))
    return pl.pallas_call(
        _node_body,
        grid=(nb,),
        in_specs=[
            pl.BlockSpec((BN, H), lambda i: (i, 0)),
            pl.BlockSpec((NC, BN, H), lambda i: (0, i, 0)),
            full((H, H)), full((H, H)), full((1, H)),
            full((H, H)), full((H, H)), full((1, H)),
        ],
        out_specs=[pl.BlockSpec((BN, H), lambda i: (i, 0))] * 3,
        out_shape=[jax.ShapeDtypeStruct((N, H), jnp.float32)] * 3,
    )(h, agg, Wn1, Wn2, b_node, Ws, Wd, b_edge)


def _dec_body(h_ref, w1_ref, b1_ref, w2_ref, b2_ref, o_ref):
    t = jnp.dot(h_ref[...], w1_ref[...],
                preferred_element_type=jnp.float32) + b1_ref[...]
    t = jnp.maximum(t, 0.0)
    o_ref[...] = jnp.dot(t, w2_ref[...],
                         preferred_element_type=jnp.float32) + b2_ref[...]


def _decode(h, W1c, b1c, W2blk, b2c, d_out):
    nb = N // BN
    full = lambda shape: pl.BlockSpec(shape, lambda i: (0, 0))
    return pl.pallas_call(
        _dec_body,
        grid=(nb,),
        in_specs=[
            pl.BlockSpec((BN, H), lambda i: (i, 0)),
            full((H, W1c.shape[1])), full((1, W1c.shape[1])),
            full((W2blk.shape[0], d_out)), full((1, d_out)),
        ],
        out_specs=pl.BlockSpec((BN, d_out), lambda i: (i, 0)),
        out_shape=jax.ShapeDtypeStruct((N, d_out), jnp.float32),
    )(h, W1c, b1c, W2blk, b2c)


def _sc_agg_body(a_hbm, b_hbm, c_hbm, src_hbm, dst_hbm, out_hbm,
                 src_g, dst_g, av0, bv0, mv0, av1, bv1, mv1, agg_sp,
                 sem, isem, ssem):
    c = lax.axis_index("c")
    s = lax.axis_index("s")
    w = c * NS + s
    ebase = w * EPW
    bufs = ((av0, bv0, mv0), (av1, bv1, mv1))

    def _zrow(r, carry):
        for j in range(LANES):
            mv0[r, pl.ds(j * 16, 16)] = jnp.zeros((16,), jnp.float32)
        return carry
    lax.fori_loop(0, KC, _zrow, 0)
    rbase = pl.multiple_of(s * RPT, 8)
    off = 0
    while off < RPT:
        n = min(KC, RPT - off)
        pltpu.sync_copy(mv0.at[pl.ds(0, n)], agg_sp.at[pl.ds(rbase + off, n)])
        off += n

    @pl.when(s == NS - 1)
    def _zero_tail():
        pltpu.sync_copy(mv0.at[pl.ds(0, RTAIL)],
                        agg_sp.at[pl.ds(NS * RPT, RTAIL)])
    plsc.subcore_barrier()

    def _group(g, carry):
        gs = pltpu.make_async_copy(src_hbm.at[w, g], src_g, isem)
        gd = pltpu.make_async_copy(dst_hbm.at[w, g], dst_g, isem)
        gs.start()
        gd.start()
        gs.wait()
        gd.wait()

        def _mk(i):
            a_, b_, m_ = bufs[i % 2]
            cb = pl.multiple_of(ebase + (g * G + i) * KC, 8)
            return (pltpu.make_async_copy(a_hbm.at[src_g.at[i]], a_, sem),
                    pltpu.make_async_copy(b_hbm.at[dst_g.at[i]], b_, sem),
                    pltpu.make_async_copy(c_hbm.at[pl.ds(cb, KC)], m_, sem))

        def _sc_start(i):
            m_ = bufs[i % 2][2]
            pltpu.async_copy(m_, agg_sp.at[dst_g.at[i]], ssem, add=True)

        def _sc_wait(i):
            m_ = bufs[i % 2][2]
            pltpu.make_async_copy(m_, agg_sp.at[dst_g.at[i]], ssem).wait()

        for d in _mk(0):
            d.start()
        for i in range(G):
            for d in _mk(i):
                d.wait()
            if i >= 1:
                _sc_wait(i - 1)
            if i + 1 < G:
                for d in _mk(i + 1):
                    d.start()
            av_, bv_, mv_ = bufs[i % 2]

            @plsc.parallel_loop(0, KC, unroll=4)
            def _row(r, av_=av_, bv_=bv_, mv_=mv_):
                for j in range(LANES):
                    sl = pl.ds(j * 16, 16)
                    mv_[r, sl] = jnp.maximum(
                        av_[r, sl] + bv_[r, sl] + mv_[r, sl], 0.0)
            _sc_start(i)
        _sc_wait(G - 1)
        return carry
    lax.fori_loop(0, NG, _group, 0)

    plsc.subcore_barrier()
    pltpu.sync_copy(agg_sp.at[pl.ds(rbase, RPT)],
                    out_hbm.at[c, pl.ds(rbase, RPT)])

    @pl.when(s == NS - 1)
    def _write_tail():
        pltpu.sync_copy(agg_sp.at[pl.ds(NS * RPT, RTAIL)],
                        out_hbm.at[c, pl.ds(NS * RPT, RTAIL)])


@functools.lru_cache(maxsize=None)
def _make_sc_agg():
    mesh = plsc.VectorSubcoreMesh(core_axis_name="c", subcore_axis_name="s",
                                  num_cores=NC, num_subcores=NS)
    return pl.kernel(
        _sc_agg_body,
        out_type=jax.ShapeDtypeStruct((NC, N, H), jnp.float32),
        mesh=mesh,
        scratch_types=[
            pltpu.VMEM((G, KC), jnp.int32),
            pltpu.VMEM((G, KC), jnp.int32),
            pltpu.VMEM((KC, H), jnp.float32),
            pltpu.VMEM((KC, H), jnp.float32),
            pltpu.VMEM((KC, H), jnp.float32),
            pltpu.VMEM((KC, H), jnp.float32),
            pltpu.VMEM((KC, H), jnp.float32),
            pltpu.VMEM((KC, H), jnp.float32),
            pltpu.VMEM_SHARED((N, H), jnp.float32),
            pltpu.SemaphoreType.DMA,
            pltpu.SemaphoreType.DMA,
            pltpu.SemaphoreType.DMA,
        ],
    )


def _sc_agg(A, B, C, src4, dst4):
    return _make_sc_agg()(A, B, C, src4, dst4)


def kernel(x, edge_index, edge_attr, layer_num, W_enc, b_enc, W_edge, b_edge,
           W_node, b_node, dec):
    Ws = W_edge[:H]
    Wd = W_edge[H:2 * H]
    We = W_edge[2 * H:]
    Wn1 = W_node[:H]
    Wn2 = W_node[H:]
    b_enc2 = b_enc.reshape(1, H)
    b_edge2 = b_edge.reshape(1, H)
    b_node2 = b_node.reshape(1, H)

    src4 = edge_index[0].astype(jnp.int32).reshape(NW, NG, G, KC)
    dst4 = edge_index[1].astype(jnp.int32).reshape(NW, NG, G, KC)

    C = _edge_c(edge_attr, We)
    h, A, B = _encode(x, W_enc, b_enc2, Ws, Wd, b_edge2)

    def _layer(_, carry):
        h, A, B = carry
        agg = _sc_agg(A, B, C, src4, dst4)
        return _node_update(h, agg, Wn1, Wn2, b_node2, Ws, Wd, b_edge2)

    h, _, _ = lax.fori_loop(0, layer_num, _layer, (h, A, B))

    names = ["dispX", "dispZ", "momentY", "momentZ", "shearY", "shearZ"]
    W1c = jnp.concatenate([dec[k][0] for k in names], axis=1)
    b1c = jnp.concatenate([dec[k][1] for k in names]).reshape(1, -1)
    W2blk = jax.scipy.linalg.block_diag(*[dec[k][2] for k in names])
    b2c = jnp.concatenate([dec[k][3] for k in names]).reshape(1, -1)
    return _decode(h, W1c, b1c, W2blk, b2c, W2blk.shape[1])
